# baseline (speedup 1.0000x reference)
import functools

import jax
import jax.numpy as jnp
from jax import lax
from jax.experimental import pallas as pl
from jax.experimental.pallas import tpu as pltpu
from jax.experimental.pallas import tpu_sc as plsc

_HIDDEN = 4096
_NUM_EXPERTS = 64
_TOP_K = 8
_BLOCK_T = 1024
_NSLICE = 4
_NWORKER = 32


def _matmul_kernel(x_ref, w_ref, logits_ref):
    logits_ref[...] = jax.lax.dot_general(
        x_ref[...], w_ref[...], (((1,), (1,)), ((), ())),
        preferred_element_type=jnp.float32,
    )


def _tc_logits(x, gate_w):
    s = x.shape[0]
    return pl.pallas_call(
        _matmul_kernel,
        grid=(s // _BLOCK_T,),
        in_specs=[
            pl.BlockSpec((_BLOCK_T, _HIDDEN), lambda i: (i, 0)),
            pl.BlockSpec((_NUM_EXPERTS, _HIDDEN), lambda i: (0, 0)),
        ],
        out_specs=pl.BlockSpec((_BLOCK_T, _NUM_EXPERTS), lambda i: (i, 0)),
        out_shape=jax.ShapeDtypeStruct((s, _NUM_EXPERTS), jnp.float32),
        compiler_params=pltpu.CompilerParams(
            dimension_semantics=("parallel",),
        ),
    )(x, gate_w)


def _merge(ak, av, bk, bv):
    rbk = lax.rev(bk, (0,))
    rbv = lax.rev(bv, (0,))
    m = ak >= rbk
    hk = jnp.where(m, ak, rbk)
    hv = jnp.where(m, av, rbv)
    return plsc.sort_key_val(hk, hv, descending=True)


def _sc_topk(logits_flat, s):
    nt = s // _NWORKER
    mesh = plsc.VectorSubcoreMesh(core_axis_name="c", subcore_axis_name="s")

    @functools.partial(
        pl.kernel,
        mesh=mesh,
        out_type=[
            jax.ShapeDtypeStruct((s * _TOP_K,), jnp.float32),
            jax.ShapeDtypeStruct((s * _TOP_K,), jnp.int32),
        ],
        scratch_types=[
            pltpu.VMEM((nt * _NUM_EXPERTS,), jnp.float32),
            pltpu.VMEM((nt * 16,), jnp.float32),
            pltpu.VMEM((nt * 16,), jnp.int32),
            pltpu.VMEM((nt * _TOP_K,), jnp.float32),
            pltpu.VMEM((nt * _TOP_K,), jnp.int32),
        ],
        compiler_params=pltpu.CompilerParams(needs_layout_passes=False),
    )
    def k(logits_hbm, topw_hbm, topi_hbm, lg_v, w_v, i_v, wc_v, ic_v):
        wid = lax.axis_index("s") * 2 + lax.axis_index("c")
        base = wid * nt
        pltpu.sync_copy(
            logits_hbm.at[pl.ds(base * _NUM_EXPERTS, nt * _NUM_EXPERTS)],
            lg_v,
        )
        io16 = lax.iota(jnp.int32, 16)
        mask8 = io16 < _TOP_K

        def body(t, carry):
            off = t * _NUM_EXPERTS
            k0 = lg_v[pl.ds(off, 16)]
            k1 = lg_v[pl.ds(off + 16, 16)]
            k2 = lg_v[pl.ds(off + 32, 16)]
            k3 = lg_v[pl.ds(off + 48, 16)]
            s0k, s0v = plsc.sort_key_val(k0, io16, descending=True)
            s1k, s1v = plsc.sort_key_val(k1, io16 + 16, descending=True)
            s2k, s2v = plsc.sort_key_val(k2, io16 + 32, descending=True)
            s3k, s3v = plsc.sort_key_val(k3, io16 + 48, descending=True)
            ak, av = _merge(s0k, s0v, s1k, s1v)
            bk, bv = _merge(s2k, s2v, s3k, s3v)
            mk, mv = _merge(ak, av, bk, bv)
            mx = jnp.max(mk)
            e = jnp.where(mask8, jnp.exp(mk - mx), jnp.float32(0.0))
            wv = e / jnp.sum(e)
            w_v[pl.ds(t * 16, 16)] = wv
            i_v[pl.ds(t * 16, 16)] = mv
            return carry

        lax.fori_loop(0, nt, body, 0)

        def compact(u, carry):
            gidx = u * 32 + io16 + jnp.where(mask8, 0, _TOP_K)
            wc_v[pl.ds(u * 16, 16)] = plsc.load_gather(w_v, [gidx])
            ic_v[pl.ds(u * 16, 16)] = plsc.load_gather(i_v, [gidx])
            return carry

        lax.fori_loop(0, nt // 2, compact, 0)
        pltpu.sync_copy(
            wc_v,
            topw_hbm.at[pl.ds(base * _TOP_K, nt * _TOP_K)],
        )
        pltpu.sync_copy(
            ic_v,
            topi_hbm.at[pl.ds(base * _TOP_K, nt * _TOP_K)],
        )

    return k(logits_flat)


@jax.jit
def kernel(hidden_states, gate_w):
    tokens = hidden_states.shape[0]
    s = tokens // _NSLICE
    logits_parts = []
    topw_parts = []
    topi_parts = []
    for i in range(_NSLICE):
        xi = jax.lax.slice_in_dim(hidden_states, i * s, (i + 1) * s, axis=0)
        lg = _tc_logits(xi, gate_w)
        logits_parts.append(lg)
        w8, i8 = _sc_topk(lg.reshape(-1), s)
        topw_parts.append(w8.reshape(s, _TOP_K))
        topi_parts.append(i8.reshape(s, _TOP_K))
    logits = jnp.concatenate(logits_parts, axis=0)
    topw = jnp.concatenate(topw_parts, axis=0)
    topi = jnp.concatenate(topi_parts, axis=0)
    return topw, topi, logits


# speedup vs baseline: 2.7258x; 2.7258x over previous
import jax
import jax.numpy as jnp
from jax.experimental import pallas as pl
from jax.experimental.pallas import tpu as pltpu

_HIDDEN = 4096
_NUM_EXPERTS = 64
_TOP_K = 8
_BLOCK_T = 1024
_CHUNK = 128


def _router_kernel(x_ref, w_ref, logits_ref, topw_ref, topi_ref):
    w = w_ref[...]
    iota_f = jax.lax.broadcasted_iota(
        jnp.int32, (_CHUNK, _NUM_EXPERTS), 1
    ).astype(jnp.float32)
    iota8 = jax.lax.broadcasted_iota(jnp.int32, (_CHUNK, _TOP_K), 1)
    neg_inf = jnp.float32(-jnp.inf)
    sentinel = jnp.float32(_NUM_EXPERTS)

    for c in range(_BLOCK_T // _CHUNK):
        rows = slice(c * _CHUNK, (c + 1) * _CHUNK)
        xc = x_ref[rows, :]
        work = jax.lax.dot_general(
            xc, w, (((1,), (1,)), ((), ())), preferred_element_type=jnp.float32
        )
        logits_ref[rows, :] = work
        vacc = jnp.zeros((_CHUNK, _TOP_K), jnp.float32)
        facc = jnp.zeros((_CHUNK, _TOP_K), jnp.float32)
        for k in range(_TOP_K):
            m = jnp.max(work, axis=1, keepdims=True)
            cand = jnp.where(work == m, iota_f, sentinel)
            idx = jnp.min(cand, axis=1, keepdims=True)
            vacc = jnp.where(iota8 == k, m, vacc)
            facc = jnp.where(iota8 == k, idx, facc)
            if k < _TOP_K - 1:
                work = jnp.where(cand == idx, neg_inf, work)
        e = jnp.exp(vacc - vacc[:, :1])
        topw_ref[rows, :] = e / jnp.sum(e, axis=1, keepdims=True)
        topi_ref[rows, :] = facc.astype(jnp.int32)


@jax.jit
def kernel(hidden_states, gate_w):
    tokens = hidden_states.shape[0]
    grid = (tokens // _BLOCK_T,)
    out_shapes = (
        jax.ShapeDtypeStruct((tokens, _NUM_EXPERTS), jnp.float32),
        jax.ShapeDtypeStruct((tokens, _TOP_K), jnp.float32),
        jax.ShapeDtypeStruct((tokens, _TOP_K), jnp.int32),
    )
    logits, topw, topi = pl.pallas_call(
        _router_kernel,
        grid=grid,
        in_specs=[
            pl.BlockSpec((_BLOCK_T, _HIDDEN), lambda i: (i, 0)),
            pl.BlockSpec((_NUM_EXPERTS, _HIDDEN), lambda i: (0, 0)),
        ],
        out_specs=[
            pl.BlockSpec((_BLOCK_T, _NUM_EXPERTS), lambda i: (i, 0)),
            pl.BlockSpec((_BLOCK_T, _TOP_K), lambda i: (i, 0)),
            pl.BlockSpec((_BLOCK_T, _TOP_K), lambda i: (i, 0)),
        ],
        out_shape=out_shapes,
        compiler_params=pltpu.CompilerParams(
            dimension_semantics=("parallel",),
        ),
    )(hidden_states, gate_w)
    return topw, topi, logits
